# Initial kernel scaffold; baseline (speedup 1.0000x reference)
import jax
import jax.numpy as jnp
from jax.experimental import pallas as pl


def kernel(e, z_img, z_ehr, z_concept, edge_index, Ws1, Wn1, b1, Ws2, Wn2, b2, Wf, bf, Wh, bh):
    raise NotImplementedError("write your pallas kernel here")


# speedup vs baseline: 5.6584x; 5.6584x over previous
import functools
import math

import jax
import jax.numpy as jnp
from jax import lax
from jax.experimental import pallas as pl
from jax.experimental.pallas import tpu as pltpu
from jax.experimental.pallas import tpu_sc as plsc

_NC = 2
_NS = 16
_CHUNK = 128
_HALF = 128

_HI = jax.lax.Precision.HIGHEST


def _dot(a, b):
  return jnp.dot(a, b, preferred_element_type=jnp.float32, precision=_HI)


def _make_segsum(n_acc, n_chunks, rpt):
  mesh = plsc.VectorSubcoreMesh(
      core_axis_name="c", subcore_axis_name="s",
      num_cores=_NC, num_subcores=_NS)

  def body(y_hbm, srcs_hbm, dsts_hbm, zeros_hbm, out_hbm,
           acc, src_v, dst_v, rows_v, sem):
    c = lax.axis_index("c")
    s = lax.axis_index("s")

    pltpu.sync_copy(zeros_hbm, acc.at[pl.ds(s * rpt, rpt)])
    plsc.subcore_barrier()

    def chunk(j, carry):
      pltpu.sync_copy(srcs_hbm.at[c, s, j], src_v)
      pltpu.sync_copy(dsts_hbm.at[s, j], dst_v)
      pltpu.async_copy(y_hbm.at[src_v], rows_v, sem).wait()
      pltpu.sync_copy(rows_v, acc.at[dst_v], add=True)
      return carry

    lax.fori_loop(0, n_chunks, chunk, 0)
    plsc.subcore_barrier()

    pltpu.sync_copy(acc.at[pl.ds(s * rpt, rpt)],
                    out_hbm.at[pl.ds(c * n_acc + s * rpt, rpt)])

  return pl.kernel(
      body,
      out_type=[jax.ShapeDtypeStruct((_NC * n_acc, _HALF), jnp.float32)],
      mesh=mesh,
      scratch_types=[
          pltpu.VMEM_SHARED((n_acc, _HALF), jnp.float32),
          pltpu.VMEM((_CHUNK,), jnp.int32),
          pltpu.VMEM((_CHUNK,), jnp.int32),
          pltpu.VMEM((_CHUNK, _HALF), jnp.float32),
          pltpu.SemaphoreType.DMA,
      ])


def _make_deg(n_acc, n_chunks, rpt):
  mesh = plsc.VectorSubcoreMesh(
      core_axis_name="c", subcore_axis_name="s",
      num_cores=_NC, num_subcores=_NS)

  def body(dsts_hbm, zeros_hbm, ones_hbm, out_hbm, acc, dst_v, ones_v):
    c = lax.axis_index("c")
    s = lax.axis_index("s")
    pltpu.sync_copy(zeros_hbm, acc.at[pl.ds(s * rpt, rpt)])
    pltpu.sync_copy(ones_hbm, ones_v)
    plsc.subcore_barrier()

    def chunk(j, carry):
      pltpu.sync_copy(dsts_hbm.at[c, s, j], dst_v)
      pltpu.sync_copy(ones_v, acc.at[dst_v], add=True)
      return carry

    lax.fori_loop(0, n_chunks, chunk, 0)
    plsc.subcore_barrier()
    pltpu.sync_copy(acc.at[pl.ds(s * rpt, rpt)],
                    out_hbm.at[pl.ds(c * n_acc + s * rpt, rpt)])

  return pl.kernel(
      body,
      out_type=[jax.ShapeDtypeStruct((_NC * n_acc, _HALF), jnp.float32)],
      mesh=mesh,
      scratch_types=[
          pltpu.VMEM_SHARED((n_acc, _HALF), jnp.float32),
          pltpu.VMEM((_CHUNK,), jnp.int32),
          pltpu.VMEM((_CHUNK, _HALF), jnp.float32),
      ])


def _tc1_body(e_ref, zi_ref, ze_ref, zc_ref, wn1_ref, ws1_ref, b1_ref,
              wfi_ref, wfe_ref, wfc_ref, bf_ref, y1_ref, s1_ref, zp_ref):
  e = e_ref[...]
  y = _dot(e, wn1_ref[...])
  y1_ref[0] = y[:, :_HALF]
  y1_ref[1] = y[:, _HALF:]
  s1_ref[...] = _dot(e, ws1_ref[...]) + b1_ref[...]
  zp_ref[...] = (_dot(zi_ref[...], wfi_ref[...])
                 + _dot(ze_ref[...], wfe_ref[...])
                 + _dot(zc_ref[...], wfc_ref[...]) + bf_ref[...])


def _tc2_body(s1_ref, agg_ref, deg_ref, wn2_ref, ws2_ref, b2_ref,
              y2_ref, s2_ref):
  deg = jnp.maximum(deg_ref[0, :, 0:1] + deg_ref[1, :, 0:1], 1.0)
  agg = jnp.concatenate([agg_ref[0], agg_ref[1]], axis=1) / deg
  h = jnp.maximum(s1_ref[...] + agg, 0.0)
  y = _dot(h, wn2_ref[...])
  y2_ref[0] = y[:, :_HALF]
  y2_ref[1] = y[:, _HALF:]
  s2_ref[...] = _dot(h, ws2_ref[...]) + b2_ref[...]


def _tc3_body(s2_ref, agg_ref, deg_ref, zp_ref, wfg_ref, wh_ref, bh_ref,
              logits_ref, haz_ref, surv_ref, hg_ref, risk_ref, *, T):
  deg = jnp.maximum(deg_ref[0, :, 0:1] + deg_ref[1, :, 0:1], 1.0)
  agg = jnp.concatenate([agg_ref[0], agg_ref[1]], axis=1) / deg
  hg = jnp.maximum(s2_ref[...] + agg, 0.0)
  hg_ref[...] = hg
  fused = jnp.maximum(_dot(hg, wfg_ref[...]) + zp_ref[...], 0.0)
  lg = _dot(fused, wh_ref[...]) + bh_ref[...]
  logits_ref[...] = lg
  hz = 1.0 / (1.0 + jnp.exp(-lg))
  haz_ref[...] = hz
  om = 1.0 - hz
  p = om[:, 0:1]
  cols = [p]
  for t in range(1, T):
    p = p * om[:, t:t + 1]
    cols.append(p)
  surv_ref[...] = jnp.concatenate(cols, axis=1)
  csum = hz[:, 0:1]
  racc = csum
  for t in range(1, T):
    csum = csum + hz[:, t:t + 1]
    racc = racc + csum
  risk_ref[...] = racc


def kernel(e, z_img, z_ehr, z_concept, edge_index, Ws1, Wn1, b1,
           Ws2, Wn2, b2, Wf, bf, Wh, bh):
  N, D = e.shape
  E = edge_index.shape[1]
  T = Wh.shape[1]
  DI = z_img.shape[1]
  DE = z_ehr.shape[1]
  DC = z_concept.shape[1]
  H = Ws1.shape[1]

  n_chunks = -(-E // (_NS * _CHUNK))
  e_pad = _NS * n_chunks * _CHUNK
  rpt = ((-(-(N + 1) // _NS) + 7) // 8) * 8
  n_acc = _NS * rpt

  src = edge_index[0]
  dst = edge_index[1]
  pad = e_pad - E
  srcs = jnp.concatenate([src, jnp.zeros((pad,), jnp.int32)]
                         ).reshape(_NS, n_chunks, _CHUNK)
  srcs = jnp.stack([srcs, srcs + N])
  dst_padded = jnp.concatenate([dst, jnp.full((pad,), N, jnp.int32)])
  dsts = dst_padded.reshape(_NS, n_chunks, _CHUNK)
  n_chunks_d = -(-E // (_NC * _NS * _CHUNK))
  e_pad_d = _NC * _NS * n_chunks_d * _CHUNK
  dsts_d = jnp.concatenate(
      [dst, jnp.full((e_pad_d - E,), N, jnp.int32)]
  ).reshape(_NC, _NS, n_chunks_d, _CHUNK)
  zeros128 = jnp.zeros((rpt, _HALF), jnp.float32)
  ones128 = jnp.ones((_CHUNK, _HALF), jnp.float32)

  bn = 1000 if N % 1000 == 0 else 8 * (-(-N // 8))
  grid = -(-N // bn)

  row_spec = lambda w: pl.BlockSpec((bn, w), lambda i: (i, 0))
  full_spec = lambda a, b: pl.BlockSpec((a, b), lambda i: (0, 0))
  split_spec = pl.BlockSpec((_NC, bn, _HALF), lambda i: (0, i, 0))

  tc1 = pl.pallas_call(
      _tc1_body,
      grid=(grid,),
      in_specs=[row_spec(D), row_spec(DI), row_spec(DE), row_spec(DC),
                full_spec(D, H), full_spec(D, H), full_spec(1, H),
                full_spec(DI, H), full_spec(DE, H), full_spec(DC, H),
                full_spec(1, H)],
      out_specs=[split_spec, row_spec(H), row_spec(H)],
      out_shape=[jax.ShapeDtypeStruct((_NC, N, _HALF), jnp.float32),
                 jax.ShapeDtypeStruct((N, H), jnp.float32),
                 jax.ShapeDtypeStruct((N, H), jnp.float32)],
  )
  y1, s1, zp = tc1(e, z_img, z_ehr, z_concept,
                   Wn1, Ws1, b1.reshape(1, H),
                   Wf[H:H + DI], Wf[H + DI:H + DI + DE],
                   Wf[H + DI + DE:], bf.reshape(1, H))

  degk = _make_deg(n_acc, n_chunks_d, rpt)
  (degp,) = degk(dsts_d, zeros128, ones128)
  degp = degp.reshape(_NC, n_acc, _HALF)

  segsum = _make_segsum(n_acc, n_chunks, rpt)
  (agg1,) = segsum(y1.reshape(_NC * N, _HALF), srcs, dsts, zeros128)
  agg1 = agg1.reshape(_NC, n_acc, _HALF)

  split_in = pl.BlockSpec((_NC, bn, _HALF), lambda i: (0, i, 0))
  tc2 = pl.pallas_call(
      _tc2_body,
      grid=(grid,),
      in_specs=[row_spec(H), split_in, split_in,
                full_spec(H, H), full_spec(H, H), full_spec(1, H)],
      out_specs=[split_spec, row_spec(H)],
      out_shape=[jax.ShapeDtypeStruct((_NC, N, _HALF), jnp.float32),
                 jax.ShapeDtypeStruct((N, H), jnp.float32)],
  )
  y2, s2 = tc2(s1, agg1, degp, Wn2, Ws2, b2.reshape(1, H))

  (agg2,) = segsum(y2.reshape(_NC * N, _HALF), srcs, dsts, zeros128)
  agg2 = agg2.reshape(_NC, n_acc, _HALF)

  tc3 = pl.pallas_call(
      functools.partial(_tc3_body, T=T),
      grid=(grid,),
      in_specs=[row_spec(H), split_in, split_in, row_spec(H),
                full_spec(H, H), full_spec(H, T), full_spec(1, T)],
      out_specs=[row_spec(T), row_spec(T), row_spec(T), row_spec(H),
                 row_spec(1)],
      out_shape=[jax.ShapeDtypeStruct((N, T), jnp.float32),
                 jax.ShapeDtypeStruct((N, T), jnp.float32),
                 jax.ShapeDtypeStruct((N, T), jnp.float32),
                 jax.ShapeDtypeStruct((N, H), jnp.float32),
                 jax.ShapeDtypeStruct((N, 1), jnp.float32)],
  )
  logits, hazards, survival, h_graph, risk = tc3(
      s2, agg2, degp, zp, Wf[:H], Wh, bh.reshape(1, T))

  return (logits, hazards, survival, h_graph, risk.reshape(N))


# speedup vs baseline: 7.0643x; 1.2485x over previous
import functools
import math

import jax
import jax.numpy as jnp
from jax import lax
from jax.experimental import pallas as pl
from jax.experimental.pallas import tpu as pltpu
from jax.experimental.pallas import tpu_sc as plsc

_NC = 2
_NS = 16
_CHUNK = 128
_HALF = 128

_HI = jax.lax.Precision.HIGHEST


def _dot(a, b):
  return jnp.dot(a, b, preferred_element_type=jnp.float32, precision=_HI)


def _make_segsum(n_acc, n_chunks, rpt):
  mesh = plsc.VectorSubcoreMesh(
      core_axis_name="c", subcore_axis_name="s",
      num_cores=_NC, num_subcores=_NS)

  n_pairs = n_chunks // 2

  def body(y_hbm, srcs_hbm, dsts_hbm, zeros_hbm, out_hbm,
           acc, src0, dst0, src1, dst1, rows0, rows1, sem):
    c = lax.axis_index("c")
    s = lax.axis_index("s")

    pltpu.sync_copy(zeros_hbm, acc.at[pl.ds(s * rpt, rpt)])
    plsc.subcore_barrier()

    def wait_rows(rows):
      pltpu.make_async_copy(y_hbm.at[src0], rows, sem).wait()

    pltpu.sync_copy(srcs_hbm.at[c, s, 0], src0)
    pltpu.sync_copy(dsts_hbm.at[s, 0], dst0)
    pltpu.async_copy(y_hbm.at[src0], rows0, sem)

    def pair(g, carry):
      a = 2 * g
      pltpu.sync_copy(srcs_hbm.at[c, s, a + 1], src1)
      pltpu.sync_copy(dsts_hbm.at[s, a + 1], dst1)
      wait_rows(rows0)
      pltpu.async_copy(y_hbm.at[src1], rows1, sem)
      pltpu.sync_copy(rows0, acc.at[dst0], add=True)
      pltpu.sync_copy(srcs_hbm.at[c, s, a + 2], src0)
      pltpu.sync_copy(dsts_hbm.at[s, a + 2], dst0)
      wait_rows(rows1)
      pltpu.async_copy(y_hbm.at[src0], rows0, sem)
      pltpu.sync_copy(rows1, acc.at[dst1], add=True)
      return carry

    lax.fori_loop(0, n_pairs - 1, pair, 0)

    a = n_chunks - 2
    pltpu.sync_copy(srcs_hbm.at[c, s, a + 1], src1)
    pltpu.sync_copy(dsts_hbm.at[s, a + 1], dst1)
    wait_rows(rows0)
    pltpu.async_copy(y_hbm.at[src1], rows1, sem)
    pltpu.sync_copy(rows0, acc.at[dst0], add=True)
    wait_rows(rows1)
    pltpu.sync_copy(rows1, acc.at[dst1], add=True)

    plsc.subcore_barrier()

    pltpu.sync_copy(acc.at[pl.ds(s * rpt, rpt)],
                    out_hbm.at[pl.ds(c * n_acc + s * rpt, rpt)])

  return pl.kernel(
      body,
      out_type=[jax.ShapeDtypeStruct((_NC * n_acc, _HALF), jnp.float32)],
      mesh=mesh,
      scratch_types=[
          pltpu.VMEM_SHARED((n_acc, _HALF), jnp.float32),
          pltpu.VMEM((_CHUNK,), jnp.int32),
          pltpu.VMEM((_CHUNK,), jnp.int32),
          pltpu.VMEM((_CHUNK,), jnp.int32),
          pltpu.VMEM((_CHUNK,), jnp.int32),
          pltpu.VMEM((_CHUNK, _HALF), jnp.float32),
          pltpu.VMEM((_CHUNK, _HALF), jnp.float32),
          pltpu.SemaphoreType.DMA,
      ])


def _make_deg(n_acc, n_chunks, rpt):
  mesh = plsc.VectorSubcoreMesh(
      core_axis_name="c", subcore_axis_name="s",
      num_cores=_NC, num_subcores=_NS)

  def body(dsts_hbm, zeros_hbm, ones_hbm, out_hbm, acc, dst_v, ones_v):
    c = lax.axis_index("c")
    s = lax.axis_index("s")
    pltpu.sync_copy(zeros_hbm, acc.at[pl.ds(s * rpt, rpt)])
    pltpu.sync_copy(ones_hbm, ones_v)
    plsc.subcore_barrier()

    def chunk(j, carry):
      pltpu.sync_copy(dsts_hbm.at[c, s, j], dst_v)
      pltpu.sync_copy(ones_v, acc.at[dst_v], add=True)
      return carry

    lax.fori_loop(0, n_chunks, chunk, 0)
    plsc.subcore_barrier()
    pltpu.sync_copy(acc.at[pl.ds(s * rpt, rpt)],
                    out_hbm.at[pl.ds(c * n_acc + s * rpt, rpt)])

  return pl.kernel(
      body,
      out_type=[jax.ShapeDtypeStruct((_NC * n_acc, _HALF), jnp.float32)],
      mesh=mesh,
      scratch_types=[
          pltpu.VMEM_SHARED((n_acc, _HALF), jnp.float32),
          pltpu.VMEM((_CHUNK,), jnp.int32),
          pltpu.VMEM((_CHUNK, _HALF), jnp.float32),
      ])


def _tc1_body(e_ref, zi_ref, ze_ref, zc_ref, wn1_ref, ws1_ref, b1_ref,
              wfi_ref, wfe_ref, wfc_ref, bf_ref, y1_ref, s1_ref, zp_ref):
  e = e_ref[...]
  y = _dot(e, wn1_ref[...])
  y1_ref[0] = y[:, :_HALF]
  y1_ref[1] = y[:, _HALF:]
  s1_ref[...] = _dot(e, ws1_ref[...]) + b1_ref[...]
  zp_ref[...] = (_dot(zi_ref[...], wfi_ref[...])
                 + _dot(ze_ref[...], wfe_ref[...])
                 + _dot(zc_ref[...], wfc_ref[...]) + bf_ref[...])


def _tc2_body(s1_ref, agg_ref, deg_ref, wn2_ref, ws2_ref, b2_ref,
              y2_ref, s2_ref):
  deg = jnp.maximum(deg_ref[0, :, 0:1] + deg_ref[1, :, 0:1], 1.0)
  agg = jnp.concatenate([agg_ref[0], agg_ref[1]], axis=1) / deg
  h = jnp.maximum(s1_ref[...] + agg, 0.0)
  y = _dot(h, wn2_ref[...])
  y2_ref[0] = y[:, :_HALF]
  y2_ref[1] = y[:, _HALF:]
  s2_ref[...] = _dot(h, ws2_ref[...]) + b2_ref[...]


def _tc3_body(s2_ref, agg_ref, deg_ref, zp_ref, wfg_ref, wh_ref, bh_ref,
              logits_ref, haz_ref, surv_ref, hg_ref, risk_ref, *, T):
  deg = jnp.maximum(deg_ref[0, :, 0:1] + deg_ref[1, :, 0:1], 1.0)
  agg = jnp.concatenate([agg_ref[0], agg_ref[1]], axis=1) / deg
  hg = jnp.maximum(s2_ref[...] + agg, 0.0)
  hg_ref[...] = hg
  fused = jnp.maximum(_dot(hg, wfg_ref[...]) + zp_ref[...], 0.0)
  lg = _dot(fused, wh_ref[...]) + bh_ref[...]
  logits_ref[...] = lg
  hz = 1.0 / (1.0 + jnp.exp(-lg))
  haz_ref[...] = hz
  om = 1.0 - hz
  p = om[:, 0:1]
  cols = [p]
  for t in range(1, T):
    p = p * om[:, t:t + 1]
    cols.append(p)
  surv_ref[...] = jnp.concatenate(cols, axis=1)
  csum = hz[:, 0:1]
  racc = csum
  for t in range(1, T):
    csum = csum + hz[:, t:t + 1]
    racc = racc + csum
  risk_ref[...] = racc


def kernel(e, z_img, z_ehr, z_concept, edge_index, Ws1, Wn1, b1,
           Ws2, Wn2, b2, Wf, bf, Wh, bh):
  N, D = e.shape
  E = edge_index.shape[1]
  T = Wh.shape[1]
  DI = z_img.shape[1]
  DE = z_ehr.shape[1]
  DC = z_concept.shape[1]
  H = Ws1.shape[1]

  n_chunks = -(-E // (_NS * _CHUNK))
  e_pad = _NS * n_chunks * _CHUNK
  rpt = ((-(-(N + 1) // _NS) + 7) // 8) * 8
  n_acc = _NS * rpt

  src = edge_index[0]
  dst = edge_index[1]
  pad = e_pad - E
  srcs = jnp.concatenate([src, jnp.zeros((pad,), jnp.int32)]
                         ).reshape(_NS, n_chunks, _CHUNK)
  srcs = jnp.stack([srcs, srcs + N])
  dst_padded = jnp.concatenate([dst, jnp.full((pad,), N, jnp.int32)])
  dsts = dst_padded.reshape(_NS, n_chunks, _CHUNK)
  n_chunks_d = -(-E // (_NC * _NS * _CHUNK))
  e_pad_d = _NC * _NS * n_chunks_d * _CHUNK
  dsts_d = jnp.concatenate(
      [dst, jnp.full((e_pad_d - E,), N, jnp.int32)]
  ).reshape(_NC, _NS, n_chunks_d, _CHUNK)
  zeros128 = jnp.zeros((rpt, _HALF), jnp.float32)
  ones128 = jnp.ones((_CHUNK, _HALF), jnp.float32)

  bn = 1000 if N % 1000 == 0 else 8 * (-(-N // 8))
  grid = -(-N // bn)

  row_spec = lambda w: pl.BlockSpec((bn, w), lambda i: (i, 0))
  full_spec = lambda a, b: pl.BlockSpec((a, b), lambda i: (0, 0))
  split_spec = pl.BlockSpec((_NC, bn, _HALF), lambda i: (0, i, 0))

  tc1 = pl.pallas_call(
      _tc1_body,
      grid=(grid,),
      in_specs=[row_spec(D), row_spec(DI), row_spec(DE), row_spec(DC),
                full_spec(D, H), full_spec(D, H), full_spec(1, H),
                full_spec(DI, H), full_spec(DE, H), full_spec(DC, H),
                full_spec(1, H)],
      out_specs=[split_spec, row_spec(H), row_spec(H)],
      out_shape=[jax.ShapeDtypeStruct((_NC, N, _HALF), jnp.float32),
                 jax.ShapeDtypeStruct((N, H), jnp.float32),
                 jax.ShapeDtypeStruct((N, H), jnp.float32)],
  )
  y1, s1, zp = tc1(e, z_img, z_ehr, z_concept,
                   Wn1, Ws1, b1.reshape(1, H),
                   Wf[H:H + DI], Wf[H + DI:H + DI + DE],
                   Wf[H + DI + DE:], bf.reshape(1, H))

  degk = _make_deg(n_acc, n_chunks_d, rpt)
  (degp,) = degk(dsts_d, zeros128, ones128)
  degp = degp.reshape(_NC, n_acc, _HALF)

  segsum = _make_segsum(n_acc, n_chunks, rpt)
  (agg1,) = segsum(y1.reshape(_NC * N, _HALF), srcs, dsts, zeros128)
  agg1 = agg1.reshape(_NC, n_acc, _HALF)

  split_in = pl.BlockSpec((_NC, bn, _HALF), lambda i: (0, i, 0))
  tc2 = pl.pallas_call(
      _tc2_body,
      grid=(grid,),
      in_specs=[row_spec(H), split_in, split_in,
                full_spec(H, H), full_spec(H, H), full_spec(1, H)],
      out_specs=[split_spec, row_spec(H)],
      out_shape=[jax.ShapeDtypeStruct((_NC, N, _HALF), jnp.float32),
                 jax.ShapeDtypeStruct((N, H), jnp.float32)],
  )
  y2, s2 = tc2(s1, agg1, degp, Wn2, Ws2, b2.reshape(1, H))

  (agg2,) = segsum(y2.reshape(_NC * N, _HALF), srcs, dsts, zeros128)
  agg2 = agg2.reshape(_NC, n_acc, _HALF)

  tc3 = pl.pallas_call(
      functools.partial(_tc3_body, T=T),
      grid=(grid,),
      in_specs=[row_spec(H), split_in, split_in, row_spec(H),
                full_spec(H, H), full_spec(H, T), full_spec(1, T)],
      out_specs=[row_spec(T), row_spec(T), row_spec(T), row_spec(H),
                 row_spec(1)],
      out_shape=[jax.ShapeDtypeStruct((N, T), jnp.float32),
                 jax.ShapeDtypeStruct((N, T), jnp.float32),
                 jax.ShapeDtypeStruct((N, T), jnp.float32),
                 jax.ShapeDtypeStruct((N, H), jnp.float32),
                 jax.ShapeDtypeStruct((N, 1), jnp.float32)],
  )
  logits, hazards, survival, h_graph, risk = tc3(
      s2, agg2, degp, zp, Wf[:H], Wh, bh.reshape(1, T))

  return (logits, hazards, survival, h_graph, risk.reshape(N))


# speedup vs baseline: 7.6623x; 1.0846x over previous
import functools
import math

import jax
import jax.numpy as jnp
from jax import lax
from jax.experimental import pallas as pl
from jax.experimental.pallas import tpu as pltpu
from jax.experimental.pallas import tpu_sc as plsc

_NC = 2
_NS = 16
_CHUNK = 128
_HALF = 128

def _dot(a, b):
  return jnp.dot(a, b, preferred_element_type=jnp.float32)


def _make_segsum(n_acc, n_chunks, rpt):
  mesh = plsc.VectorSubcoreMesh(
      core_axis_name="c", subcore_axis_name="s",
      num_cores=_NC, num_subcores=_NS)

  n_pairs = n_chunks // 2

  def body(y_hbm, srcs_hbm, dsts_hbm, zeros_hbm, out_hbm,
           acc, src0, dst0, src1, dst1, rows0, rows1, sem):
    c = lax.axis_index("c")
    s = lax.axis_index("s")

    pltpu.sync_copy(zeros_hbm, acc.at[pl.ds(s * rpt, rpt)])
    plsc.subcore_barrier()

    def wait_rows(rows):
      pltpu.make_async_copy(y_hbm.at[src0], rows, sem).wait()

    pltpu.sync_copy(srcs_hbm.at[c, s, 0], src0)
    pltpu.sync_copy(dsts_hbm.at[s, 0], dst0)
    pltpu.async_copy(y_hbm.at[src0], rows0, sem)

    def pair(g, carry):
      a = 2 * g
      pltpu.sync_copy(srcs_hbm.at[c, s, a + 1], src1)
      pltpu.sync_copy(dsts_hbm.at[s, a + 1], dst1)
      wait_rows(rows0)
      pltpu.async_copy(y_hbm.at[src1], rows1, sem)
      pltpu.sync_copy(rows0, acc.at[dst0], add=True)
      pltpu.sync_copy(srcs_hbm.at[c, s, a + 2], src0)
      pltpu.sync_copy(dsts_hbm.at[s, a + 2], dst0)
      wait_rows(rows1)
      pltpu.async_copy(y_hbm.at[src0], rows0, sem)
      pltpu.sync_copy(rows1, acc.at[dst1], add=True)
      return carry

    lax.fori_loop(0, n_pairs - 1, pair, 0)

    a = n_chunks - 2
    pltpu.sync_copy(srcs_hbm.at[c, s, a + 1], src1)
    pltpu.sync_copy(dsts_hbm.at[s, a + 1], dst1)
    wait_rows(rows0)
    pltpu.async_copy(y_hbm.at[src1], rows1, sem)
    pltpu.sync_copy(rows0, acc.at[dst0], add=True)
    wait_rows(rows1)
    pltpu.sync_copy(rows1, acc.at[dst1], add=True)

    plsc.subcore_barrier()

    pltpu.sync_copy(acc.at[pl.ds(s * rpt, rpt)],
                    out_hbm.at[pl.ds(c * n_acc + s * rpt, rpt)])

  return pl.kernel(
      body,
      out_type=[jax.ShapeDtypeStruct((_NC * n_acc, _HALF), jnp.float32)],
      mesh=mesh,
      scratch_types=[
          pltpu.VMEM_SHARED((n_acc, _HALF), jnp.float32),
          pltpu.VMEM((_CHUNK,), jnp.int32),
          pltpu.VMEM((_CHUNK,), jnp.int32),
          pltpu.VMEM((_CHUNK,), jnp.int32),
          pltpu.VMEM((_CHUNK,), jnp.int32),
          pltpu.VMEM((_CHUNK, _HALF), jnp.float32),
          pltpu.VMEM((_CHUNK, _HALF), jnp.float32),
          pltpu.SemaphoreType.DMA,
      ])


def _make_deg(n_acc, n_chunks, rpt):
  mesh = plsc.VectorSubcoreMesh(
      core_axis_name="c", subcore_axis_name="s",
      num_cores=_NC, num_subcores=_NS)

  def body(dsts_hbm, zeros_hbm, ones_hbm, out_hbm, acc, dst_v, ones_v):
    c = lax.axis_index("c")
    s = lax.axis_index("s")
    pltpu.sync_copy(zeros_hbm, acc.at[pl.ds(s * rpt, rpt)])
    pltpu.sync_copy(ones_hbm, ones_v)
    plsc.subcore_barrier()

    def chunk(j, carry):
      pltpu.sync_copy(dsts_hbm.at[c, s, j], dst_v)
      pltpu.sync_copy(ones_v, acc.at[dst_v], add=True)
      return carry

    lax.fori_loop(0, n_chunks, chunk, 0)
    plsc.subcore_barrier()
    pltpu.sync_copy(acc.at[pl.ds(s * rpt, rpt)],
                    out_hbm.at[pl.ds(c * n_acc + s * rpt, rpt)])

  return pl.kernel(
      body,
      out_type=[jax.ShapeDtypeStruct((_NC * n_acc, _HALF), jnp.float32)],
      mesh=mesh,
      scratch_types=[
          pltpu.VMEM_SHARED((n_acc, _HALF), jnp.float32),
          pltpu.VMEM((_CHUNK,), jnp.int32),
          pltpu.VMEM((_CHUNK, _HALF), jnp.float32),
      ])


def _tc1_body(e_ref, zi_ref, ze_ref, zc_ref, wn1_ref, ws1_ref, b1_ref,
              wfi_ref, wfe_ref, wfc_ref, bf_ref, y1_ref, s1_ref, zp_ref):
  e = e_ref[...]
  y = _dot(e, wn1_ref[...])
  y1_ref[0] = y[:, :_HALF]
  y1_ref[1] = y[:, _HALF:]
  s1_ref[...] = _dot(e, ws1_ref[...]) + b1_ref[...]
  zp_ref[...] = (_dot(zi_ref[...], wfi_ref[...])
                 + _dot(ze_ref[...], wfe_ref[...])
                 + _dot(zc_ref[...], wfc_ref[...]) + bf_ref[...])


def _tc2_body(s1_ref, agg_ref, deg_ref, wn2_ref, ws2_ref, b2_ref,
              y2_ref, s2_ref):
  deg = jnp.maximum(deg_ref[0, :, 0:1] + deg_ref[1, :, 0:1], 1.0)
  agg = jnp.concatenate([agg_ref[0], agg_ref[1]], axis=1) / deg
  h = jnp.maximum(s1_ref[...] + agg, 0.0)
  y = _dot(h, wn2_ref[...])
  y2_ref[0] = y[:, :_HALF]
  y2_ref[1] = y[:, _HALF:]
  s2_ref[...] = _dot(h, ws2_ref[...]) + b2_ref[...]


def _tc3_body(s2_ref, agg_ref, deg_ref, zp_ref, wfg_ref, wh_ref, bh_ref,
              logits_ref, haz_ref, surv_ref, hg_ref, risk_ref, *, T):
  deg = jnp.maximum(deg_ref[0, :, 0:1] + deg_ref[1, :, 0:1], 1.0)
  agg = jnp.concatenate([agg_ref[0], agg_ref[1]], axis=1) / deg
  hg = jnp.maximum(s2_ref[...] + agg, 0.0)
  hg_ref[...] = hg
  fused = jnp.maximum(_dot(hg, wfg_ref[...]) + zp_ref[...], 0.0)
  lg = _dot(fused, wh_ref[...]) + bh_ref[...]
  logits_ref[...] = lg
  hz = 1.0 / (1.0 + jnp.exp(-lg))
  haz_ref[...] = hz
  om = 1.0 - hz
  p = om[:, 0:1]
  cols = [p]
  for t in range(1, T):
    p = p * om[:, t:t + 1]
    cols.append(p)
  surv_ref[...] = jnp.concatenate(cols, axis=1)
  csum = hz[:, 0:1]
  racc = csum
  for t in range(1, T):
    csum = csum + hz[:, t:t + 1]
    racc = racc + csum
  risk_ref[...] = racc


def kernel(e, z_img, z_ehr, z_concept, edge_index, Ws1, Wn1, b1,
           Ws2, Wn2, b2, Wf, bf, Wh, bh):
  N, D = e.shape
  E = edge_index.shape[1]
  T = Wh.shape[1]
  DI = z_img.shape[1]
  DE = z_ehr.shape[1]
  DC = z_concept.shape[1]
  H = Ws1.shape[1]

  n_chunks = -(-E // (_NS * _CHUNK))
  e_pad = _NS * n_chunks * _CHUNK
  rpt = ((-(-(N + 1) // _NS) + 7) // 8) * 8
  n_acc = _NS * rpt

  src = edge_index[0]
  dst = edge_index[1]
  pad = e_pad - E
  srcs = jnp.concatenate([src, jnp.zeros((pad,), jnp.int32)]
                         ).reshape(_NS, n_chunks, _CHUNK)
  srcs = jnp.stack([srcs, srcs + N])
  dst_padded = jnp.concatenate([dst, jnp.full((pad,), N, jnp.int32)])
  dsts = dst_padded.reshape(_NS, n_chunks, _CHUNK)
  n_chunks_d = -(-E // (_NC * _NS * _CHUNK))
  e_pad_d = _NC * _NS * n_chunks_d * _CHUNK
  dsts_d = jnp.concatenate(
      [dst, jnp.full((e_pad_d - E,), N, jnp.int32)]
  ).reshape(_NC, _NS, n_chunks_d, _CHUNK)
  zeros128 = jnp.zeros((rpt, _HALF), jnp.float32)
  ones128 = jnp.ones((_CHUNK, _HALF), jnp.float32)

  bn = 1000 if N % 1000 == 0 else 8 * (-(-N // 8))
  grid = -(-N // bn)

  row_spec = lambda w: pl.BlockSpec((bn, w), lambda i: (i, 0))
  full_spec = lambda a, b: pl.BlockSpec((a, b), lambda i: (0, 0))
  split_spec = pl.BlockSpec((_NC, bn, _HALF), lambda i: (0, i, 0))

  tc1 = pl.pallas_call(
      _tc1_body,
      grid=(grid,),
      in_specs=[row_spec(D), row_spec(DI), row_spec(DE), row_spec(DC),
                full_spec(D, H), full_spec(D, H), full_spec(1, H),
                full_spec(DI, H), full_spec(DE, H), full_spec(DC, H),
                full_spec(1, H)],
      out_specs=[split_spec, row_spec(H), row_spec(H)],
      out_shape=[jax.ShapeDtypeStruct((_NC, N, _HALF), jnp.float32),
                 jax.ShapeDtypeStruct((N, H), jnp.float32),
                 jax.ShapeDtypeStruct((N, H), jnp.float32)],
  )
  y1, s1, zp = tc1(e, z_img, z_ehr, z_concept,
                   Wn1, Ws1, b1.reshape(1, H),
                   Wf[H:H + DI], Wf[H + DI:H + DI + DE],
                   Wf[H + DI + DE:], bf.reshape(1, H))

  degk = _make_deg(n_acc, n_chunks_d, rpt)
  (degp,) = degk(dsts_d, zeros128, ones128)
  degp = degp.reshape(_NC, n_acc, _HALF)

  segsum = _make_segsum(n_acc, n_chunks, rpt)
  (agg1,) = segsum(y1.reshape(_NC * N, _HALF), srcs, dsts, zeros128)
  agg1 = agg1.reshape(_NC, n_acc, _HALF)

  split_in = pl.BlockSpec((_NC, bn, _HALF), lambda i: (0, i, 0))
  tc2 = pl.pallas_call(
      _tc2_body,
      grid=(grid,),
      in_specs=[row_spec(H), split_in, split_in,
                full_spec(H, H), full_spec(H, H), full_spec(1, H)],
      out_specs=[split_spec, row_spec(H)],
      out_shape=[jax.ShapeDtypeStruct((_NC, N, _HALF), jnp.float32),
                 jax.ShapeDtypeStruct((N, H), jnp.float32)],
  )
  y2, s2 = tc2(s1, agg1, degp, Wn2, Ws2, b2.reshape(1, H))

  (agg2,) = segsum(y2.reshape(_NC * N, _HALF), srcs, dsts, zeros128)
  agg2 = agg2.reshape(_NC, n_acc, _HALF)

  tc3 = pl.pallas_call(
      functools.partial(_tc3_body, T=T),
      grid=(grid,),
      in_specs=[row_spec(H), split_in, split_in, row_spec(H),
                full_spec(H, H), full_spec(H, T), full_spec(1, T)],
      out_specs=[row_spec(T), row_spec(T), row_spec(T), row_spec(H),
                 row_spec(1)],
      out_shape=[jax.ShapeDtypeStruct((N, T), jnp.float32),
                 jax.ShapeDtypeStruct((N, T), jnp.float32),
                 jax.ShapeDtypeStruct((N, T), jnp.float32),
                 jax.ShapeDtypeStruct((N, H), jnp.float32),
                 jax.ShapeDtypeStruct((N, 1), jnp.float32)],
  )
  logits, hazards, survival, h_graph, risk = tc3(
      s2, agg2, degp, zp, Wf[:H], Wh, bh.reshape(1, T))

  return (logits, hazards, survival, h_graph, risk.reshape(N))
